# baseline (speedup 1.0000x reference)
import functools

import jax
import jax.numpy as jnp
from jax import lax
from jax.experimental import pallas as pl
from jax.experimental.pallas import tpu as pltpu
from jax.experimental.pallas import tpu_sc as plsc

FIELDS = 26
VOCAB = 100000
BATCH = 16384
EMB = 16
D = FIELDS * EMB
H1 = 64
H2 = 32
EPS = 1e-5

NIDX = BATCH * FIELDS
NW = 32
G = 128
PER_W = NIDX // NW
BPW = BATCH // NW
CH_G = 8
CH = CH_G * G
NCH = PER_W // CH

def _sc_body(tab2, tab1, xiT, deep_out, first_out,
             xi_v, idx2_v, idx1_v, rows_v, vals_v, acc_v, gsem, vsem):
    wid = lax.axis_index("s") * 2 + lax.axis_index("c")
    for i in range(FIELDS):
        pltpu.sync_copy(xiT.at[i, pl.ds(wid * BPW, BPW)],
                        xi_v.at[pl.ds(i * BPW, BPW)])
    lane = jnp.arange(16, dtype=jnp.int32)
    stride26 = lane * FIELDS
    for i in range(FIELDS):
        a2 = (i // 8) * (VC8 * 8) + (i % 8)
        a1 = i * VC8

        def grp(g, _, i=i, a2=a2, a1=a1):
            x = xi_v[pl.ds(i * BPW + g * 16, 16)]
            p = stride26 + (g * 16 * FIELDS + i)
            plsc.store_scatter(idx2_v, [p], x * 8 + a2)
            plsc.store_scatter(idx1_v, [p], x + a1)
            return 0

        lax.fori_loop(0, BPW // 16, grp, 0)

    def chunk(c, acc):
        rcopies = []
        vcopies = []
        for g in range(CH_G):
            row = c * CH_G + g
            rcopies.append(pltpu.make_async_copy(
                tab2.at[idx2_v.at[pl.ds(row * G, G)]],
                rows_v.at[pl.ds(g * G, G)], gsem))
            vcopies.append(pltpu.make_async_copy(
                tab1.at[idx1_v.at[pl.ds(row * G, G)]],
                vals_v.at[pl.ds(g * G, G)], vsem))
        for cp in rcopies:
            cp.start()
        for cp in vcopies:
            cp.start()
        for cp in rcopies:
            cp.wait()
        for cp in vcopies:
            cp.wait()
        pltpu.sync_copy(rows_v, deep_out.at[pl.ds(wid * PER_W + c * CH, CH)])

        def accum(j, a):
            return a + vals_v[pl.ds(j * 16, 16)]
        return lax.fori_loop(0, CH // 16, accum, acc)

    acc = lax.fori_loop(0, NCH, chunk, jnp.zeros((16,), jnp.float32))
    acc_v[...] = acc
    pltpu.sync_copy(acc_v, first_out.at[wid])


@functools.lru_cache(maxsize=None)
def _sc_gather():
    mesh = plsc.VectorSubcoreMesh(core_axis_name="c", subcore_axis_name="s")
    return pl.kernel(
        _sc_body,
        out_type=[
            jax.ShapeDtypeStruct((NIDX, EMB), jnp.float32),
            jax.ShapeDtypeStruct((NW, 16), jnp.float32),
        ],
        mesh=mesh,
        compiler_params=pltpu.CompilerParams(use_tc_tiling_on_sc=False,
                                             needs_layout_passes=False),
        scratch_types=[
            pltpu.VMEM((FIELDS * BPW,), jnp.int32),
            pltpu.VMEM((PER_W,), jnp.int32),
            pltpu.VMEM((PER_W,), jnp.int32),
            pltpu.VMEM((CH, EMB), jnp.float32),
            pltpu.VMEM((CH,), jnp.float32),
            pltpu.VMEM((16,), jnp.float32),
            pltpu.SemaphoreType.DMA,
            pltpu.SemaphoreType.DMA,
        ],
    )


TR_G = 4
TR_VC = 13312
TR_NC = 8
VC8 = TR_VC * TR_NC


def _tr_body(x_ref, x1_ref, o_ref, o1_ref):
    x = x_ref[...].reshape(8 * EMB, TR_VC)
    o_ref[...] = jnp.transpose(x)
    o1_ref[...] = x1_ref[...].reshape(8, 1, TR_VC // 128, 128)


def _tr_call(emb2t, emb1t):
    return pl.pallas_call(
        _tr_body,
        grid=(TR_G, TR_NC),
        in_specs=[
            pl.BlockSpec((8, EMB, TR_VC), lambda g, c: (g, 0, c)),
            pl.BlockSpec((8, 1, TR_VC), lambda g, c: (g, 0, c)),
        ],
        out_specs=[
            pl.BlockSpec((TR_VC, 128), lambda g, c: (g * TR_NC + c, 0)),
            pl.BlockSpec((8, 1, TR_VC // 128, 128),
                         lambda g, c: (g, c, 0, 0)),
        ],
        out_shape=[
            jax.ShapeDtypeStruct((TR_G * VC8, 128), jnp.float32),
            jax.ShapeDtypeStruct((TR_G * 8, TR_NC, TR_VC // 128, 128),
                                 jnp.float32),
        ],
    )(emb2t, emb1t)


BLK = 4096


def _tc_body(deep_ref, xv_ref, e_ref, s_ref, w1_ref, b1_ref, g1_ref, be1_ref,
             w2_ref, b2_ref, g2_ref, be2_ref, fp_ref, bias_ref, out_ref):
    f32 = jnp.float32
    deep_raw = deep_ref[...]
    xv = xv_ref[...]
    scaled = deep_raw * jnp.dot(xv, e_ref[...], preferred_element_type=f32)
    fm_sum = jnp.dot(scaled, s_ref[...], preferred_element_type=f32)
    fm2 = 0.5 * (jnp.sum(fm_sum * fm_sum, axis=1)
                 - jnp.sum(scaled * scaled, axis=1))
    inv = (1.0 + EPS) ** -0.5
    h = jnp.dot(scaled, w1_ref[...], preferred_element_type=f32) + b1_ref[...]
    h = jnp.maximum(h, 0.0) * (inv * g1_ref[...]) + be1_ref[...]
    h = jnp.dot(h, w2_ref[...], preferred_element_type=f32) + b2_ref[...]
    h = jnp.maximum(h, 0.0) * (inv * g2_ref[...]) + be2_ref[...]
    dsum = jnp.sum(h, axis=1)
    first = jnp.sum(fp_ref[...])
    out_ref[...] = fm2 + dsum + (first + bias_ref[0, 0])


def _tc_call(deep, xv, e, s, w1, b1, g1, be1, w2, b2, g2, be2, fparts, bias):
    full = lambda shape: pl.BlockSpec(shape, lambda i: (0,) * len(shape))
    return pl.pallas_call(
        _tc_body,
        grid=(BATCH // BLK,),
        in_specs=[
            pl.BlockSpec((BLK, D), lambda i: (i, 0)),
            pl.BlockSpec((BLK, FIELDS), lambda i: (i, 0)),
            full((FIELDS, D)), full((D, EMB)),
            full((D, H1)), full((1, H1)), full((1, H1)), full((1, H1)),
            full((H1, H2)), full((1, H2)), full((1, H2)), full((1, H2)),
            full((NW, 16)), full((1, 1)),
        ],
        out_specs=pl.BlockSpec((BLK,), lambda i: (i,)),
        out_shape=jax.ShapeDtypeStruct((BATCH,), jnp.float32),
    )(deep, xv, e, s, w1, b1, g1, be1, w2, b2, g2, be2, fparts, bias)


def kernel(Xi, Xv, emb1, emb2, W1, b1, g1, beta1, W2, b2, g2, beta2, bias):
    t2, t1 = _tr_call(emb2.transpose(0, 2, 1), emb1.transpose(0, 2, 1))
    tab2 = t2.reshape(TR_G * VC8 * 8, EMB)
    tab1 = t1.reshape(TR_G * 8 * VC8)
    xiT = Xi.transpose(1, 2, 0).reshape(FIELDS, BATCH).astype(jnp.int32)
    deep_raw, fparts = _sc_gather()(tab2, tab1, xiT)
    f = jnp.arange(FIELDS, dtype=jnp.int32)
    jj = jnp.arange(D, dtype=jnp.int32)
    e = (jj[None, :] // EMB == f[:, None]).astype(jnp.float32)
    s = (jj[:, None] % EMB == jnp.arange(EMB)[None, :]).astype(jnp.float32)
    return _tc_call(
        deep_raw.reshape(BATCH, D), Xv, e, s,
        W1, b1.reshape(1, H1), g1.reshape(1, H1), beta1.reshape(1, H1),
        W2, b2.reshape(1, H2), g2.reshape(1, H2), beta2.reshape(1, H2),
        fparts, bias.reshape(1, 1))


# speedup vs baseline: 1.0132x; 1.0132x over previous
import functools

import jax
import jax.numpy as jnp
from jax import lax
from jax.experimental import pallas as pl
from jax.experimental.pallas import tpu as pltpu
from jax.experimental.pallas import tpu_sc as plsc

FIELDS = 26
VOCAB = 100000
BATCH = 16384
EMB = 16
D = FIELDS * EMB
H1 = 64
H2 = 32
EPS = 1e-5

NIDX = BATCH * FIELDS
NW = 32
G = 128
PER_W = NIDX // NW
BPW = BATCH // NW
CH_G = 8
CH = CH_G * G
NCH = PER_W // CH

def _sc_body(tab2, tab1, xiT, deep_out, first_out,
             xi_v, idx2_v, idx1_v, rows_v, vals_v, acc_v, gsem, vsem):
    wid = lax.axis_index("s") * 2 + lax.axis_index("c")
    for i in range(FIELDS):
        pltpu.sync_copy(xiT.at[i, pl.ds(wid * BPW, BPW)],
                        xi_v.at[pl.ds(i * BPW, BPW)])
    lane = jnp.arange(16, dtype=jnp.int32)
    stride26 = lane * FIELDS
    for i in range(FIELDS):
        a2 = (i // 8) * (VC8 * 8) + (i % 8)
        a1 = i * VC8

        def grp(g, _, i=i, a2=a2, a1=a1):
            x = xi_v[pl.ds(i * BPW + g * 16, 16)]
            p = stride26 + (g * 16 * FIELDS + i)
            plsc.store_scatter(idx2_v, [p], x * 8 + a2)
            plsc.store_scatter(idx1_v, [p], x + a1)
            return 0

        lax.fori_loop(0, BPW // 16, grp, 0)

    def chunk(c, acc):
        rcopies = []
        vcopies = []
        for g in range(CH_G):
            row = c * CH_G + g
            rcopies.append(pltpu.make_async_copy(
                tab2.at[idx2_v.at[pl.ds(row * G, G)]],
                rows_v.at[pl.ds(g * G, G)], gsem))
            vcopies.append(pltpu.make_async_copy(
                tab1.at[idx1_v.at[pl.ds(row * G, G)]],
                vals_v.at[pl.ds(g * G, G)], vsem))
        for cp in rcopies:
            cp.start()
        for cp in vcopies:
            cp.start()
        for cp in rcopies:
            cp.wait()
        for cp in vcopies:
            cp.wait()
        pltpu.sync_copy(rows_v, deep_out.at[pl.ds(wid * PER_W + c * CH, CH)])

        def accum(j, a):
            return a + vals_v[pl.ds(j * 16, 16)]
        return lax.fori_loop(0, CH // 16, accum, acc)

    acc = lax.fori_loop(0, NCH, chunk, jnp.zeros((16,), jnp.float32))
    acc_v[...] = acc
    pltpu.sync_copy(acc_v, first_out.at[wid])


@functools.lru_cache(maxsize=None)
def _sc_gather():
    mesh = plsc.VectorSubcoreMesh(core_axis_name="c", subcore_axis_name="s")
    return pl.kernel(
        _sc_body,
        out_type=[
            jax.ShapeDtypeStruct((NIDX, EMB), jnp.float32),
            jax.ShapeDtypeStruct((NW, 16), jnp.float32),
        ],
        mesh=mesh,
        compiler_params=pltpu.CompilerParams(use_tc_tiling_on_sc=False,
                                             needs_layout_passes=False),
        scratch_types=[
            pltpu.VMEM((FIELDS * BPW,), jnp.int32),
            pltpu.VMEM((PER_W,), jnp.int32),
            pltpu.VMEM((PER_W,), jnp.int32),
            pltpu.VMEM((CH, EMB), jnp.float32),
            pltpu.VMEM((CH,), jnp.float32),
            pltpu.VMEM((16,), jnp.float32),
            pltpu.SemaphoreType.DMA,
            pltpu.SemaphoreType.DMA,
        ],
    )


TR_G = 4
TR_VC = 13312
TR_NC = 8
VC8 = TR_VC * TR_NC


def _tr_body(x_ref, x1_ref, o_ref, o1_ref):
    x = x_ref[...].reshape(8 * EMB, TR_VC)
    o_ref[...] = jnp.transpose(x)
    o1_ref[...] = x1_ref[...].reshape(8, 1, TR_VC // 128, 128)


def _tr_call(emb2t, emb1t):
    return pl.pallas_call(
        _tr_body,
        grid=(TR_G, TR_NC),
        in_specs=[
            pl.BlockSpec((8, EMB, TR_VC), lambda g, c: (g, 0, c)),
            pl.BlockSpec((8, 1, TR_VC), lambda g, c: (g, 0, c)),
        ],
        out_specs=[
            pl.BlockSpec((TR_VC, 128), lambda g, c: (g * TR_NC + c, 0)),
            pl.BlockSpec((8, 1, TR_VC // 128, 128),
                         lambda g, c: (g, c, 0, 0)),
        ],
        out_shape=[
            jax.ShapeDtypeStruct((TR_G * VC8, 128), jnp.float32),
            jax.ShapeDtypeStruct((TR_G * 8, TR_NC, TR_VC // 128, 128),
                                 jnp.float32),
        ],
    )(emb2t, emb1t)


BLK = 2048


def _tc_body(deep_ref, xv_ref, e_ref, s_ref, w1_ref, b1_ref, g1_ref, be1_ref,
             w2_ref, b2_ref, g2_ref, be2_ref, fp_ref, bias_ref, out_ref):
    f32 = jnp.float32
    deep_raw = deep_ref[...]
    xv = xv_ref[...]
    scaled = deep_raw * jnp.dot(xv, e_ref[...], preferred_element_type=f32)
    fm_sum = jnp.dot(scaled, s_ref[...], preferred_element_type=f32)
    fm2 = 0.5 * (jnp.sum(fm_sum * fm_sum, axis=1)
                 - jnp.sum(scaled * scaled, axis=1))
    inv = (1.0 + EPS) ** -0.5
    h = jnp.dot(scaled, w1_ref[...], preferred_element_type=f32) + b1_ref[...]
    h = jnp.maximum(h, 0.0) * (inv * g1_ref[...]) + be1_ref[...]
    h = jnp.dot(h, w2_ref[...], preferred_element_type=f32) + b2_ref[...]
    h = jnp.maximum(h, 0.0) * (inv * g2_ref[...]) + be2_ref[...]
    dsum = jnp.sum(h, axis=1)
    first = jnp.sum(fp_ref[...])
    out_ref[...] = fm2 + dsum + (first + bias_ref[0, 0])


def _tc_call(deep, xv, e, s, w1, b1, g1, be1, w2, b2, g2, be2, fparts, bias):
    full = lambda shape: pl.BlockSpec(shape, lambda i: (0,) * len(shape))
    return pl.pallas_call(
        _tc_body,
        grid=(BATCH // BLK,),
        in_specs=[
            pl.BlockSpec((BLK, D), lambda i: (i, 0)),
            pl.BlockSpec((BLK, FIELDS), lambda i: (i, 0)),
            full((FIELDS, D)), full((D, EMB)),
            full((D, H1)), full((1, H1)), full((1, H1)), full((1, H1)),
            full((H1, H2)), full((1, H2)), full((1, H2)), full((1, H2)),
            full((NW, 16)), full((1, 1)),
        ],
        out_specs=pl.BlockSpec((BLK,), lambda i: (i,)),
        out_shape=jax.ShapeDtypeStruct((BATCH,), jnp.float32),
    )(deep, xv, e, s, w1, b1, g1, be1, w2, b2, g2, be2, fparts, bias)


def kernel(Xi, Xv, emb1, emb2, W1, b1, g1, beta1, W2, b2, g2, beta2, bias):
    t2, t1 = _tr_call(emb2.transpose(0, 2, 1), emb1.transpose(0, 2, 1))
    tab2 = t2.reshape(TR_G * VC8 * 8, EMB)
    tab1 = t1.reshape(TR_G * 8 * VC8)
    xiT = Xi.transpose(1, 2, 0).reshape(FIELDS, BATCH).astype(jnp.int32)
    deep_raw, fparts = _sc_gather()(tab2, tab1, xiT)
    f = jnp.arange(FIELDS, dtype=jnp.int32)
    jj = jnp.arange(D, dtype=jnp.int32)
    e = (jj[None, :] // EMB == f[:, None]).astype(jnp.float32)
    s = (jj[:, None] % EMB == jnp.arange(EMB)[None, :]).astype(jnp.float32)
    return _tc_call(
        deep_raw.reshape(BATCH, D), Xv, e, s,
        W1, b1.reshape(1, H1), g1.reshape(1, H1), beta1.reshape(1, H1),
        W2, b2.reshape(1, H2), g2.reshape(1, H2), beta2.reshape(1, H2),
        fparts, bias.reshape(1, 1))
